# baseline (speedup 1.0000x reference)
import functools

import jax
import jax.numpy as jnp
from jax import lax
from jax.experimental import pallas as pl
from jax.experimental.pallas import tpu as pltpu
from jax.experimental.pallas import tpu_sc as plsc

N_NODES = 10000
N_EDGES = 160000
D_FEAT = 256
N_HIDDEN = 512
N_CLASSES = 7

NC, NS = 2, 16
NW = NC * NS
NPAD = 10240
EPAD = 163840
EPW = EPAD // NW
CHW = 128
NCHUNK = EPW // CHW
NBUF = 4
NGRP = NCHUNK // NBUF
NGRP2 = NCHUNK // (2 * NBUF)
NQ = 4
QF = N_HIDDEN // NQ
NH = 8
HF = N_HIDDEN // NH
RPS = NPAD // NS
DUMP_ROW = N_NODES + 100

_mesh = plsc.VectorSubcoreMesh(
    core_axis_name="c", subcore_axis_name="s", num_cores=NC, num_subcores=NS)


def _sc_deg_body(dst_hbm, zeros_hbm, ones_hbm, deg_hbm, idxv, onesv, slab, sem):
    c = lax.axis_index("c")
    s = lax.axis_index("s")
    w = c * NS + s
    pltpu.sync_copy(dst_hbm.at[w], idxv)
    pltpu.sync_copy(ones_hbm, onesv)
    pltpu.sync_copy(zeros_hbm.at[pl.ds(s * RPS, RPS)],
                    slab.at[pl.ds(s * RPS, RPS)])
    plsc.subcore_barrier()

    def chunk(j, carry):
        pltpu.async_copy(onesv, slab.at[idxv.at[j]], sem, add=True).wait()
        return carry

    lax.fori_loop(0, NCHUNK, chunk, 0)
    plsc.subcore_barrier()
    for cc in range(NC):
        @pl.when(c == cc)
        def _(cc=cc):
            pltpu.sync_copy(slab.at[pl.ds(s * RPS, RPS)],
                            deg_hbm.at[cc].at[pl.ds(s * RPS, RPS)])


_sc_deg = pl.kernel(
    _sc_deg_body,
    out_type=jax.ShapeDtypeStruct((NC, NPAD), jnp.float32),
    mesh=_mesh,
    scratch_types=[
        pltpu.VMEM((NCHUNK, CHW), jnp.int32),
        pltpu.VMEM((CHW,), jnp.float32),
        pltpu.VMEM_SHARED((NPAD,), jnp.float32),
        pltpu.SemaphoreType.DMA,
    ],
)


def _sc_spmm_body(g_hbm, src_hbm, dst_hbm, zeros_hbm, s_hbm,
                  srcv, dstv, buf, gstage, slab, sem_g, sem_s, sem_x,
                  sem_o, sem_h):
    c = lax.axis_index("c")
    s = lax.axis_index("s")
    w = c * NS + s
    rows = pl.ds(s * RPS, RPS)
    pltpu.sync_copy(src_hbm.at[w], srcv)
    pltpu.sync_copy(dst_hbm.at[w], dstv)

    for p in range(NQ):
        for cc in range(NC):
            @pl.when(c == cc)
            def _(qh=2 * p + cc):
                pltpu.async_copy(g_hbm.at[qh].at[rows], gstage.at[rows],
                                 sem_x)
        if p > 0:
            pltpu.make_async_copy(slab.at[rows], s_hbm.at[0].at[rows],
                                  sem_o).wait()
        pltpu.async_copy(zeros_hbm.at[rows], slab.at[rows], sem_x)
        pltpu.make_async_copy(g_hbm.at[0].at[rows], gstage.at[rows],
                              sem_x).wait()
        pltpu.make_async_copy(zeros_hbm.at[rows], slab.at[rows],
                              sem_x).wait()
        plsc.subcore_barrier()
        for cc in range(NC):
            @pl.when(c == cc)
            def _(qh=2 * p + cc):
                ghbm = g_hbm.at[qh]
                for b in range(NBUF):
                    pltpu.async_copy(gstage.at[srcv.at[b]], buf.at[b],
                                     sem_g)

                def pair(gp, carry):
                    for sub in range(2):
                        base = gp * 2 * NBUF + sub * NBUF
                        for b in range(NBUF):
                            j = base + b
                            is_h = sub == 1 and b == NBUF - 1
                            pltpu.make_async_copy(
                                (ghbm if is_h else gstage).at[srcv.at[j]],
                                buf.at[b],
                                sem_h if is_h else sem_g).wait()
                            pltpu.async_copy(buf.at[b],
                                             slab.at[dstv.at[j]],
                                             sem_s, add=True)

                        def refill(base=base, sub=sub):
                            for b in range(NBUF):
                                j2 = base + NBUF + b
                                is_h2 = sub == 0 and b == NBUF - 1
                                pltpu.make_async_copy(
                                    buf.at[b], slab.at[dstv.at[j2]],
                                    sem_s).wait()
                                pltpu.async_copy(
                                    (ghbm if is_h2 else gstage
                                     ).at[srcv.at[j2]], buf.at[b],
                                    sem_h if is_h2 else sem_g)

                        if sub == 0:
                            refill()
                        else:
                            pl.when(gp < NGRP2 - 1)(refill)
                    return carry

                lax.fori_loop(0, NGRP2, pair, 0)
                for b in range(NBUF):
                    pltpu.make_async_copy(buf.at[b], slab.at[dstv.at[0]],
                                          sem_s).wait()
        plsc.subcore_barrier()
        for cc in range(NC):
            @pl.when(c == cc)
            def _(qh=2 * p + cc):
                pltpu.async_copy(slab.at[rows], s_hbm.at[qh].at[rows],
                                 sem_o)
    pltpu.make_async_copy(slab.at[rows], s_hbm.at[0].at[rows], sem_o).wait()


_sc_spmm = pl.kernel(
    _sc_spmm_body,
    out_type=jax.ShapeDtypeStruct((NH, NPAD, HF), jnp.float32),
    mesh=_mesh,
    scratch_types=[
        pltpu.VMEM((NCHUNK, CHW), jnp.int32),
        pltpu.VMEM((NCHUNK, CHW), jnp.int32),
        pltpu.VMEM((NBUF, CHW, HF), jnp.float32),
        pltpu.VMEM_SHARED((NPAD, HF), jnp.float32),
        pltpu.VMEM_SHARED((NPAD, HF), jnp.float32),
        pltpu.SemaphoreType.DMA,
        pltpu.SemaphoreType.DMA,
        pltpu.SemaphoreType.DMA,
        pltpu.SemaphoreType.DMA,
        pltpu.SemaphoreType.DMA,
    ],
    compiler_params=pltpu.CompilerParams(use_tc_tiling_on_sc=False),
)


R1 = 1024


def _tc1_body(x_ref, w_ref, deg_ref, dinv_ref, g_ref):
    deg = deg_ref[:, 0:1] + deg_ref[:, 1:2] + 1.0
    dinv = lax.rsqrt(jnp.maximum(deg, 1.0))
    dinv_ref[...] = dinv
    h = jnp.dot(x_ref[...], w_ref[...], preferred_element_type=jnp.float32)
    g = h * dinv
    g_ref[0] = g[:, :HF]
    g_ref[1] = g[:, HF:]


_tc1 = pl.pallas_call(
    _tc1_body,
    grid=(NPAD // R1, NQ),
    in_specs=[
        pl.BlockSpec((R1, D_FEAT), lambda i, q: (i, 0)),
        pl.BlockSpec((D_FEAT, QF), lambda i, q: (0, q)),
        pl.BlockSpec((R1, NC), lambda i, q: (i, 0)),
    ],
    out_specs=[
        pl.BlockSpec((R1, 1), lambda i, q: (i, 0)),
        pl.BlockSpec((2, R1, HF), lambda i, q: (q, i, 0)),
    ],
    out_shape=[
        jax.ShapeDtypeStruct((NPAD, 1), jnp.float32),
        jax.ShapeDtypeStruct((NH, NPAD, HF), jnp.float32),
    ],
)


R2 = 512


def _tc23_body(s_ref, g_ref, dinv_ref, b_ref, w_ref, out_ref):
    dinv = dinv_ref[...]
    z = None
    for q in range(NQ):
        s_q = jnp.concatenate([s_ref[2 * q], s_ref[2 * q + 1]], axis=1)
        g_q = jnp.concatenate([g_ref[2 * q], g_ref[2 * q + 1]], axis=1)
        y_q = jnp.maximum(dinv * (s_q + g_q) + b_ref[q], 0.0)
        pq = jnp.dot(y_q, w_ref[pl.ds(q * QF, QF), :],
                     preferred_element_type=jnp.float32)
        z = pq if z is None else z + pq
    gnew = z * dinv
    for qh in range(NH):
        out_ref[qh] = gnew[:, qh * HF:(qh + 1) * HF]


_tc23 = pl.pallas_call(
    _tc23_body,
    grid=(NPAD // R2,),
    in_specs=[
        pl.BlockSpec((NH, R2, HF), lambda i: (0, i, 0)),
        pl.BlockSpec((NH, R2, HF), lambda i: (0, i, 0)),
        pl.BlockSpec((R2, 1), lambda i: (i, 0)),
        pl.BlockSpec((NQ, 1, QF), lambda i: (0, 0, 0)),
        pl.BlockSpec((N_HIDDEN, N_HIDDEN), lambda i: (0, 0)),
    ],
    out_specs=[
        pl.BlockSpec((NH, R2, HF), lambda i: (0, i, 0)),
    ],
    out_shape=[jax.ShapeDtypeStruct((NH, NPAD, HF), jnp.float32)],
)


R4 = 512


def _tc4_body(s_ref, g_ref, dinv_ref, b3_ref, wf1_ref, bf1_ref,
              wf2_ref, bf2_ref, wf3_ref, bf3_ref, out_ref):
    dinv = dinv_ref[...]
    z = None
    for q in range(NQ):
        s_q = jnp.concatenate([s_ref[2 * q], s_ref[2 * q + 1]], axis=1)
        g_q = jnp.concatenate([g_ref[2 * q], g_ref[2 * q + 1]], axis=1)
        y_q = jnp.maximum(dinv * (s_q + g_q) + b3_ref[q], 0.0)
        pq = jnp.dot(y_q, wf1_ref[pl.ds(q * QF, QF), :],
                     preferred_element_type=jnp.float32)
        z = pq if z is None else z + pq
    z1 = jnp.maximum(z + bf1_ref[...], 0.0)
    z2 = jnp.maximum(
        jnp.dot(z1, wf2_ref[...], preferred_element_type=jnp.float32)
        + bf2_ref[...], 0.0)
    o = (jnp.dot(z2, wf3_ref[...], preferred_element_type=jnp.float32)
         + bf3_ref[...])
    mask = lax.broadcasted_iota(jnp.int32, o.shape, 1) < N_CLASSES
    om = jnp.where(mask, o, jnp.float32(-1e30))
    m = jnp.max(om, axis=1, keepdims=True)
    ex = jnp.where(mask, jnp.exp(o - m), 0.0)
    lse = jnp.log(jnp.sum(ex, axis=1, keepdims=True))
    out_ref[...] = o - m - lse


_tc4 = pl.pallas_call(
    _tc4_body,
    grid=(NPAD // R4,),
    in_specs=[
        pl.BlockSpec((NH, R4, HF), lambda i: (0, i, 0)),
        pl.BlockSpec((NH, R4, HF), lambda i: (0, i, 0)),
        pl.BlockSpec((R4, 1), lambda i: (i, 0)),
        pl.BlockSpec((NQ, 1, QF), lambda i: (0, 0, 0)),
        pl.BlockSpec((N_HIDDEN, N_HIDDEN), lambda i: (0, 0)),
        pl.BlockSpec((1, N_HIDDEN), lambda i: (0, 0)),
        pl.BlockSpec((N_HIDDEN, N_HIDDEN), lambda i: (0, 0)),
        pl.BlockSpec((1, N_HIDDEN), lambda i: (0, 0)),
        pl.BlockSpec((N_HIDDEN, QF), lambda i: (0, 0)),
        pl.BlockSpec((1, QF), lambda i: (0, 0)),
    ],
    out_specs=[pl.BlockSpec((R4, QF), lambda i: (i, 0))],
    out_shape=[jax.ShapeDtypeStruct((NPAD, QF), jnp.float32)],
)


def kernel(x, edge_index, TRAIN, W1, b1, W2, b2, W3, b3,
           Wf1, bf1, Wf2, bf2, Wf3, bf3):
    del TRAIN
    pad_e = EPAD - N_EDGES
    srcp = jnp.concatenate(
        [edge_index[0], jnp.zeros((pad_e,), jnp.int32)]).reshape(NW, NCHUNK, CHW)
    dstp = jnp.concatenate(
        [edge_index[1], jnp.full((pad_e,), DUMP_ROW, jnp.int32)]
    ).reshape(NW, NCHUNK, CHW)
    xp = jnp.pad(x, ((0, NPAD - N_NODES), (0, 0)))
    zeros_row = jnp.zeros((NPAD,), jnp.float32)
    ones_row = jnp.ones((CHW,), jnp.float32)
    zeros_slab = jnp.zeros((NPAD, HF), jnp.float32)

    deg_parts = _sc_deg(dstp, zeros_row, ones_row)
    degT = deg_parts.T

    def spmm(g):
        return _sc_spmm(g, srcp, dstp, zeros_slab)

    dinv, g1 = _tc1(xp, W1, degT)
    s1 = spmm(g1)
    (g2,) = _tc23(s1, g1, dinv, b1.reshape(NQ, 1, QF), W2)
    s2 = spmm(g2)
    (g3,) = _tc23(s2, g2, dinv, b2.reshape(NQ, 1, QF), W3)
    s3 = spmm(g3)

    wf3p = jnp.pad(Wf3, ((0, 0), (0, QF - N_CLASSES)))
    bf3p = jnp.pad(bf3, (0, QF - N_CLASSES)).reshape(1, QF)
    (o,) = _tc4(s3, g3, dinv, b3.reshape(NQ, 1, QF),
                Wf1, bf1.reshape(1, N_HIDDEN),
                Wf2, bf2.reshape(1, N_HIDDEN), wf3p, bf3p)
    return o[:N_NODES, :N_CLASSES]


# speedup vs baseline: 1.1063x; 1.1063x over previous
import functools

import jax
import jax.numpy as jnp
from jax import lax
from jax.experimental import pallas as pl
from jax.experimental.pallas import tpu as pltpu
from jax.experimental.pallas import tpu_sc as plsc

N_NODES = 10000
N_EDGES = 160000
D_FEAT = 256
N_HIDDEN = 512
N_CLASSES = 7

NC, NS = 2, 16
NW = NC * NS
NPAD = 10240
EPAD = 163840
EPW = EPAD // NW
CHW = 128
NCHUNK = EPW // CHW
NBUF = 4
NGRP = NCHUNK // NBUF
NQ = 4
QF = N_HIDDEN // NQ
NH = 8
HF = N_HIDDEN // NH
RPS = NPAD // NS
DUMP_ROW = N_NODES + 100

_mesh = plsc.VectorSubcoreMesh(
    core_axis_name="c", subcore_axis_name="s", num_cores=NC, num_subcores=NS)


def _sc_deg_body(dst_hbm, zeros_hbm, ones_hbm, deg_hbm, idxv, onesv, slab, sem):
    c = lax.axis_index("c")
    s = lax.axis_index("s")
    w = c * NS + s
    pltpu.sync_copy(dst_hbm.at[w], idxv)
    pltpu.sync_copy(ones_hbm, onesv)
    pltpu.sync_copy(zeros_hbm.at[pl.ds(s * RPS, RPS)],
                    slab.at[pl.ds(s * RPS, RPS)])
    plsc.subcore_barrier()

    def chunk(j, carry):
        pltpu.async_copy(onesv, slab.at[idxv.at[j]], sem, add=True).wait()
        return carry

    lax.fori_loop(0, NCHUNK, chunk, 0)
    plsc.subcore_barrier()
    for cc in range(NC):
        @pl.when(c == cc)
        def _(cc=cc):
            pltpu.sync_copy(slab.at[pl.ds(s * RPS, RPS)],
                            deg_hbm.at[cc].at[pl.ds(s * RPS, RPS)])


_sc_deg = pl.kernel(
    _sc_deg_body,
    out_type=jax.ShapeDtypeStruct((NC, NPAD), jnp.float32),
    mesh=_mesh,
    scratch_types=[
        pltpu.VMEM((NCHUNK, CHW), jnp.int32),
        pltpu.VMEM((CHW,), jnp.float32),
        pltpu.VMEM_SHARED((NPAD,), jnp.float32),
        pltpu.SemaphoreType.DMA,
    ],
)


def _sc_spmm_body(g_hbm, src_hbm, dst_hbm, zeros_hbm, s_hbm,
                  srcv, dstv, buf, gstage, slab, sem_g, sem_s, sem_x, sem_o):
    c = lax.axis_index("c")
    s = lax.axis_index("s")
    w = c * NS + s
    rows = pl.ds(s * RPS, RPS)
    pltpu.sync_copy(src_hbm.at[w], srcv)
    pltpu.sync_copy(dst_hbm.at[w], dstv)

    for p in range(NQ):
        for cc in range(NC):
            @pl.when(c == cc)
            def _(qh=2 * p + cc):
                pltpu.async_copy(g_hbm.at[qh].at[rows], gstage.at[rows],
                                 sem_x)
        if p > 0:
            pltpu.make_async_copy(slab.at[rows], s_hbm.at[0].at[rows],
                                  sem_o).wait()
        pltpu.async_copy(zeros_hbm.at[rows], slab.at[rows], sem_x)
        pltpu.make_async_copy(g_hbm.at[0].at[rows], gstage.at[rows],
                              sem_x).wait()
        pltpu.make_async_copy(zeros_hbm.at[rows], slab.at[rows],
                              sem_x).wait()
        plsc.subcore_barrier()
        gq = gstage
        for b in range(NBUF):
            pltpu.async_copy(gq.at[srcv.at[b]], buf.at[b], sem_g)

        def group(g, carry):
            base = g * NBUF
            for b in range(NBUF):
                j = base + b
                pltpu.make_async_copy(
                    gq.at[srcv.at[j]], buf.at[b], sem_g).wait()
                pltpu.async_copy(buf.at[b], slab.at[dstv.at[j]],
                                 sem_s, add=True)
            @pl.when(g < NGRP - 1)
            def _():
                for b in range(NBUF):
                    j2 = base + NBUF + b
                    pltpu.make_async_copy(
                        buf.at[b], slab.at[dstv.at[j2]], sem_s).wait()
                    pltpu.async_copy(gq.at[srcv.at[j2]], buf.at[b], sem_g)
            return carry

        lax.fori_loop(0, NGRP, group, 0)
        for b in range(NBUF):
            pltpu.make_async_copy(buf.at[b], slab.at[dstv.at[0]],
                                  sem_s).wait()
        plsc.subcore_barrier()
        for cc in range(NC):
            @pl.when(c == cc)
            def _(qh=2 * p + cc):
                pltpu.async_copy(slab.at[rows], s_hbm.at[qh].at[rows],
                                 sem_o)
    pltpu.make_async_copy(slab.at[rows], s_hbm.at[0].at[rows], sem_o).wait()


_sc_spmm = pl.kernel(
    _sc_spmm_body,
    out_type=jax.ShapeDtypeStruct((NH, NPAD, HF), jnp.float32),
    mesh=_mesh,
    scratch_types=[
        pltpu.VMEM((NCHUNK, CHW), jnp.int32),
        pltpu.VMEM((NCHUNK, CHW), jnp.int32),
        pltpu.VMEM((NBUF, CHW, HF), jnp.float32),
        pltpu.VMEM_SHARED((NPAD, HF), jnp.float32),
        pltpu.VMEM_SHARED((NPAD, HF), jnp.float32),
        pltpu.SemaphoreType.DMA,
        pltpu.SemaphoreType.DMA,
        pltpu.SemaphoreType.DMA,
        pltpu.SemaphoreType.DMA,
    ],
    compiler_params=pltpu.CompilerParams(use_tc_tiling_on_sc=False),
)


R1 = 1024


def _tc1_body(x_ref, w_ref, deg_ref, dinv_ref, g_ref):
    deg = deg_ref[:, 0:1] + deg_ref[:, 1:2] + 1.0
    dinv = lax.rsqrt(jnp.maximum(deg, 1.0))
    dinv_ref[...] = dinv
    h = jnp.dot(x_ref[...], w_ref[...], preferred_element_type=jnp.float32)
    g = h * dinv
    g_ref[0] = g[:, :HF]
    g_ref[1] = g[:, HF:]


_tc1 = pl.pallas_call(
    _tc1_body,
    grid=(NPAD // R1, NQ),
    in_specs=[
        pl.BlockSpec((R1, D_FEAT), lambda i, q: (i, 0)),
        pl.BlockSpec((D_FEAT, QF), lambda i, q: (0, q)),
        pl.BlockSpec((R1, NC), lambda i, q: (i, 0)),
    ],
    out_specs=[
        pl.BlockSpec((R1, 1), lambda i, q: (i, 0)),
        pl.BlockSpec((2, R1, HF), lambda i, q: (q, i, 0)),
    ],
    out_shape=[
        jax.ShapeDtypeStruct((NPAD, 1), jnp.float32),
        jax.ShapeDtypeStruct((NH, NPAD, HF), jnp.float32),
    ],
)


R2 = 512


def _tc23_body(s_ref, g_ref, dinv_ref, b_ref, w_ref, out_ref):
    dinv = dinv_ref[...]
    z = None
    for q in range(NQ):
        s_q = jnp.concatenate([s_ref[2 * q], s_ref[2 * q + 1]], axis=1)
        g_q = jnp.concatenate([g_ref[2 * q], g_ref[2 * q + 1]], axis=1)
        y_q = jnp.maximum(dinv * (s_q + g_q) + b_ref[q], 0.0)
        pq = jnp.dot(y_q, w_ref[pl.ds(q * QF, QF), :],
                     preferred_element_type=jnp.float32)
        z = pq if z is None else z + pq
    gnew = z * dinv
    for qh in range(NH):
        out_ref[qh] = gnew[:, qh * HF:(qh + 1) * HF]


_tc23 = pl.pallas_call(
    _tc23_body,
    grid=(NPAD // R2,),
    in_specs=[
        pl.BlockSpec((NH, R2, HF), lambda i: (0, i, 0)),
        pl.BlockSpec((NH, R2, HF), lambda i: (0, i, 0)),
        pl.BlockSpec((R2, 1), lambda i: (i, 0)),
        pl.BlockSpec((NQ, 1, QF), lambda i: (0, 0, 0)),
        pl.BlockSpec((N_HIDDEN, N_HIDDEN), lambda i: (0, 0)),
    ],
    out_specs=[
        pl.BlockSpec((NH, R2, HF), lambda i: (0, i, 0)),
    ],
    out_shape=[jax.ShapeDtypeStruct((NH, NPAD, HF), jnp.float32)],
)


R4 = 512


def _tc4_body(s_ref, g_ref, dinv_ref, b3_ref, wf1_ref, bf1_ref,
              wf2_ref, bf2_ref, wf3_ref, bf3_ref, out_ref):
    dinv = dinv_ref[...]
    z = None
    for q in range(NQ):
        s_q = jnp.concatenate([s_ref[2 * q], s_ref[2 * q + 1]], axis=1)
        g_q = jnp.concatenate([g_ref[2 * q], g_ref[2 * q + 1]], axis=1)
        y_q = jnp.maximum(dinv * (s_q + g_q) + b3_ref[q], 0.0)
        pq = jnp.dot(y_q, wf1_ref[pl.ds(q * QF, QF), :],
                     preferred_element_type=jnp.float32)
        z = pq if z is None else z + pq
    z1 = jnp.maximum(z + bf1_ref[...], 0.0)
    z2 = jnp.maximum(
        jnp.dot(z1, wf2_ref[...], preferred_element_type=jnp.float32)
        + bf2_ref[...], 0.0)
    o = (jnp.dot(z2, wf3_ref[...], preferred_element_type=jnp.float32)
         + bf3_ref[...])
    mask = lax.broadcasted_iota(jnp.int32, o.shape, 1) < N_CLASSES
    om = jnp.where(mask, o, jnp.float32(-1e30))
    m = jnp.max(om, axis=1, keepdims=True)
    ex = jnp.where(mask, jnp.exp(o - m), 0.0)
    lse = jnp.log(jnp.sum(ex, axis=1, keepdims=True))
    out_ref[...] = o - m - lse


_tc4 = pl.pallas_call(
    _tc4_body,
    grid=(NPAD // R4,),
    in_specs=[
        pl.BlockSpec((NH, R4, HF), lambda i: (0, i, 0)),
        pl.BlockSpec((NH, R4, HF), lambda i: (0, i, 0)),
        pl.BlockSpec((R4, 1), lambda i: (i, 0)),
        pl.BlockSpec((NQ, 1, QF), lambda i: (0, 0, 0)),
        pl.BlockSpec((N_HIDDEN, N_HIDDEN), lambda i: (0, 0)),
        pl.BlockSpec((1, N_HIDDEN), lambda i: (0, 0)),
        pl.BlockSpec((N_HIDDEN, N_HIDDEN), lambda i: (0, 0)),
        pl.BlockSpec((1, N_HIDDEN), lambda i: (0, 0)),
        pl.BlockSpec((N_HIDDEN, QF), lambda i: (0, 0)),
        pl.BlockSpec((1, QF), lambda i: (0, 0)),
    ],
    out_specs=[pl.BlockSpec((R4, QF), lambda i: (i, 0))],
    out_shape=[jax.ShapeDtypeStruct((NPAD, QF), jnp.float32)],
)


def kernel(x, edge_index, TRAIN, W1, b1, W2, b2, W3, b3,
           Wf1, bf1, Wf2, bf2, Wf3, bf3):
    del TRAIN
    pad_e = EPAD - N_EDGES
    srcp = jnp.concatenate(
        [edge_index[0], jnp.zeros((pad_e,), jnp.int32)]).reshape(NW, NCHUNK, CHW)
    dstp = jnp.concatenate(
        [edge_index[1], jnp.full((pad_e,), DUMP_ROW, jnp.int32)]
    ).reshape(NW, NCHUNK, CHW)
    xp = jnp.pad(x, ((0, NPAD - N_NODES), (0, 0)))
    zeros_row = jnp.zeros((NPAD,), jnp.float32)
    ones_row = jnp.ones((CHW,), jnp.float32)
    zeros_slab = jnp.zeros((NPAD, HF), jnp.float32)

    deg_parts = _sc_deg(dstp, zeros_row, ones_row)
    degT = deg_parts.T

    def spmm(g):
        return _sc_spmm(g, srcp, dstp, zeros_slab)

    dinv, g1 = _tc1(xp, W1, degT)
    s1 = spmm(g1)
    (g2,) = _tc23(s1, g1, dinv, b1.reshape(NQ, 1, QF), W2)
    s2 = spmm(g2)
    (g3,) = _tc23(s2, g2, dinv, b2.reshape(NQ, 1, QF), W3)
    s3 = spmm(g3)

    wf3p = jnp.pad(Wf3, ((0, 0), (0, QF - N_CLASSES)))
    bf3p = jnp.pad(bf3, (0, QF - N_CLASSES)).reshape(1, QF)
    (o,) = _tc4(s3, g3, dinv, b3.reshape(NQ, 1, QF),
                Wf1, bf1.reshape(1, N_HIDDEN),
                Wf2, bf2.reshape(1, N_HIDDEN), wf3p, bf3p)
    return o[:N_NODES, :N_CLASSES]
